# baseline (device time: 12623 ns/iter reference)
import jax
import jax.numpy as jnp
from jax import lax
from jax.experimental import pallas as pl
from jax.experimental.pallas import tpu as pltpu

N_DEV = 4
EPS = 1e-5


def kernel(x, Wp):
    b, s_per, hw, c = x.shape
    n_out = Wp.shape[1]
    n_global = N_DEV * s_per * hw
    rows = s_per * hw // 2

    x2 = x.reshape(b, rows, 2 * c)
    z = jnp.zeros_like(Wp)
    w_blk = jnp.concatenate(
        [jnp.concatenate([Wp, z], axis=1),
         jnp.concatenate([z, Wp], axis=1)], axis=0)

    def body(x_ref, wp_ref, out_ref, comm_ref, send_sems, recv_sems):
        my = lax.axis_index("i")
        peers = [lax.rem(my + d, N_DEV) for d in range(1, N_DEV)]

        barrier_sem = pltpu.get_barrier_semaphore()
        for nbr in peers:
            pl.semaphore_signal(
                barrier_sem, inc=1,
                device_id=(nbr,), device_id_type=pl.DeviceIdType.MESH,
            )

        xp = x_ref[...]
        sp1 = jnp.sum(xp, axis=1)
        sp2 = jnp.sum(xp * xp, axis=1)
        s1 = sp1[:, :c] + sp1[:, c:]
        s2 = sp2[:, :c] + sp2[:, c:]

        pl.semaphore_wait(barrier_sem, N_DEV - 1)
        comm_ref[0, :, :] = jnp.concatenate([s1, s2], axis=0)

        rdmas = []
        for d in range(1, N_DEV):
            rdma = pltpu.make_async_remote_copy(
                src_ref=comm_ref.at[0],
                dst_ref=comm_ref.at[d],
                send_sem=send_sems.at[d - 1],
                recv_sem=recv_sems.at[d - 1],
                device_id=(peers[d - 1],),
                device_id_type=pl.DeviceIdType.MESH,
            )
            rdma.start()
            rdmas.append(rdma)
        for rdma in rdmas:
            rdma.wait_recv()

        total = (comm_ref[0, :, :] + comm_ref[1, :, :]
                 + comm_ref[2, :, :] + comm_ref[3, :, :])
        mean = total[0:2, :] / n_global
        ex2 = total[2:4, :] / n_global
        var = ex2 - mean * mean
        rstd = lax.rsqrt(var + EPS)
        mean2 = jnp.concatenate([mean, mean], axis=1)
        rstd2 = jnp.concatenate([rstd, rstd], axis=1)

        hv = (xp - mean2[:, None, :]) * rstd2[:, None, :]
        a = hv * lax.logistic(hv)
        y = jnp.dot(
            a.reshape(b * rows, 2 * c), wp_ref[...],
            preferred_element_type=jnp.float32,
        )
        out_ref[...] = y.reshape(b, rows, 2 * n_out)

        for rdma in rdmas:
            rdma.wait_send()

    y_packed = pl.pallas_call(
        body,
        out_shape=jax.ShapeDtypeStruct((b, rows, 2 * n_out), jnp.float32),
        in_specs=[
            pl.BlockSpec(memory_space=pltpu.VMEM),
            pl.BlockSpec(memory_space=pltpu.VMEM),
        ],
        out_specs=pl.BlockSpec(memory_space=pltpu.VMEM),
        scratch_shapes=[
            pltpu.VMEM((N_DEV, 4, c), jnp.float32),
            pltpu.SemaphoreType.DMA((N_DEV - 1,)),
            pltpu.SemaphoreType.DMA((N_DEV - 1,)),
        ],
        compiler_params=pltpu.CompilerParams(collective_id=0),
    )(x2, w_blk)
    return y_packed.reshape(b, s_per, hw, n_out)


# device time: 12248 ns/iter; 1.0306x vs baseline; 1.0306x over previous
import jax
import jax.numpy as jnp
from jax import lax
from jax.experimental import pallas as pl
from jax.experimental.pallas import tpu as pltpu

N_DEV = 4
EPS = 1e-5


def kernel(x, Wp):
    b, s_per, hw, c = x.shape
    n_out = Wp.shape[1]
    n_global = N_DEV * s_per * hw
    rows = s_per * hw // 2

    x2 = x.reshape(b, rows, 2 * c)
    z = jnp.zeros_like(Wp)
    w_blk = jnp.concatenate(
        [jnp.concatenate([Wp, z], axis=1),
         jnp.concatenate([z, Wp], axis=1)], axis=0)

    def body(x_ref, wp_ref, out_ref, comm_ref, send_sems, recv_sems):
        my = lax.axis_index("i")
        peers = [lax.rem(my + d, N_DEV) for d in range(1, N_DEV)]

        barrier_sem = pltpu.get_barrier_semaphore()
        for nbr in peers:
            pl.semaphore_signal(
                barrier_sem, inc=1,
                device_id=(nbr,), device_id_type=pl.DeviceIdType.MESH,
            )

        xp = x_ref[...]
        sp1 = jnp.sum(xp, axis=1)
        sp2 = jnp.sum(xp * xp, axis=1)
        s1 = sp1[:, :c] + sp1[:, c:]
        s2 = sp2[:, :c] + sp2[:, c:]

        pl.semaphore_wait(barrier_sem, N_DEV - 1)
        comm_ref[0, :, :] = jnp.concatenate([s1, s2], axis=0)

        rdmas = []
        for d in range(1, N_DEV):
            rdma = pltpu.make_async_remote_copy(
                src_ref=comm_ref.at[0],
                dst_ref=comm_ref.at[d],
                send_sem=send_sems.at[d - 1],
                recv_sem=recv_sems.at[d - 1],
                device_id=(peers[d - 1],),
                device_id_type=pl.DeviceIdType.MESH,
            )
            rdma.start()
            rdmas.append(rdma)
        for rdma in rdmas:
            rdma.wait_recv()

        total = (comm_ref[0, :, :] + comm_ref[1, :, :]
                 + comm_ref[2, :, :] + comm_ref[3, :, :])
        mean = total[0:2, :] / n_global
        ex2 = total[2:4, :] / n_global
        var = ex2 - mean * mean
        rstd = lax.rsqrt(var + EPS)
        mean2 = jnp.concatenate([mean, mean], axis=1)
        rstd2 = jnp.concatenate([rstd, rstd], axis=1)

        hv = (xp - mean2[:, None, :]) * rstd2[:, None, :]
        a = hv * lax.logistic(hv)
        y = jnp.dot(
            a.reshape(b * rows, 2 * c), wp_ref[...],
            preferred_element_type=jnp.float32,
        )
        out_ref[...] = y.reshape(b, s_per, hw, n_out)

        for rdma in rdmas:
            rdma.wait_send()

    y_packed = pl.pallas_call(
        body,
        out_shape=jax.ShapeDtypeStruct((b, s_per, hw, n_out), jnp.float32),
        in_specs=[
            pl.BlockSpec(memory_space=pltpu.VMEM),
            pl.BlockSpec(memory_space=pltpu.VMEM),
        ],
        out_specs=pl.BlockSpec(memory_space=pltpu.VMEM),
        scratch_shapes=[
            pltpu.VMEM((N_DEV, 4, c), jnp.float32),
            pltpu.SemaphoreType.DMA((N_DEV - 1,)),
            pltpu.SemaphoreType.DMA((N_DEV - 1,)),
        ],
        compiler_params=pltpu.CompilerParams(collective_id=0),
    )(x2, w_blk)
    return y_packed


# device time: 11934 ns/iter; 1.0577x vs baseline; 1.0263x over previous
import jax
import jax.numpy as jnp
from jax import lax
from jax.experimental import pallas as pl
from jax.experimental.pallas import tpu as pltpu

N_DEV = 4
EPS = 1e-5


def kernel(x, Wp):
    b, s_per, hw, c = x.shape
    n_out = Wp.shape[1]
    n_global = N_DEV * s_per * hw

    def body(x_ref, wp_ref, out_ref, comm_ref, send_sems, recv_sems):
        my = lax.axis_index("i")
        peers = [lax.rem(my + d, N_DEV) for d in range(1, N_DEV)]

        barrier_sem = pltpu.get_barrier_semaphore()
        for nbr in peers:
            pl.semaphore_signal(
                barrier_sem, inc=1,
                device_id=(nbr,), device_id_type=pl.DeviceIdType.MESH,
            )

        xv = x_ref[...].reshape(b, s_per * hw, c)
        s1 = jnp.sum(xv, axis=1)
        s2 = jnp.sum(xv * xv, axis=1)

        pl.semaphore_wait(barrier_sem, N_DEV - 1)
        comm_ref[0, :, :] = jnp.concatenate([s1, s2], axis=0)

        rdmas = []
        for d in range(1, N_DEV):
            rdma = pltpu.make_async_remote_copy(
                src_ref=comm_ref.at[0],
                dst_ref=comm_ref.at[d],
                send_sem=send_sems.at[d - 1],
                recv_sem=recv_sems.at[d - 1],
                device_id=(peers[d - 1],),
                device_id_type=pl.DeviceIdType.MESH,
            )
            rdma.start()
            rdmas.append(rdma)
        for rdma in rdmas:
            rdma.wait_recv()

        total = (comm_ref[0, :, :] + comm_ref[1, :, :]
                 + comm_ref[2, :, :] + comm_ref[3, :, :])
        mean = total[0:2, :] / n_global
        ex2 = total[2:4, :] / n_global
        var = ex2 - mean * mean
        rstd = lax.rsqrt(var + EPS)

        hv = (xv - mean[:, None, :]) * rstd[:, None, :]
        a = hv * lax.logistic(hv)
        y = jnp.dot(
            a.reshape(b * s_per * hw, c), wp_ref[...],
            preferred_element_type=jnp.float32,
        )
        out_ref[...] = y.reshape(b, s_per, hw, n_out)

        for rdma in rdmas:
            rdma.wait_send()

    return pl.pallas_call(
        body,
        out_shape=jax.ShapeDtypeStruct((b, s_per, hw, n_out), jnp.float32),
        in_specs=[
            pl.BlockSpec(memory_space=pltpu.VMEM),
            pl.BlockSpec(memory_space=pltpu.VMEM),
        ],
        out_specs=pl.BlockSpec(memory_space=pltpu.VMEM),
        scratch_shapes=[
            pltpu.VMEM((N_DEV, 4, c), jnp.float32),
            pltpu.SemaphoreType.DMA((N_DEV - 1,)),
            pltpu.SemaphoreType.DMA((N_DEV - 1,)),
        ],
        compiler_params=pltpu.CompilerParams(collective_id=0),
    )(x, Wp)
